# baseline (speedup 1.0000x reference)
import dataclasses
import functools

import jax
import jax.numpy as jnp
from jax import lax
from jax.experimental import pallas as pl
from jax.experimental.pallas import tpu as pltpu
from jax.experimental.pallas import tpu_sc as plsc

N_NODES = 10000
N_EDGES = 160000
D = 256
HALF = 128
NEG_SLOPE = 0.2

NC = 2
NS = 16
CHUNK = 128
NCHUNK = 84
RING = 3
IRING = 2
STEP = 6
EDGES_PAD = NS * NCHUNK * CHUNK
ROWS_PER_SUB = 632
ROWS_LAST = N_NODES - 15 * ROWS_PER_SUB

BLK = 1000
GRID = N_NODES // BLK


def _seg_sum_sc(h2, epack):
    mesh = plsc.VectorSubcoreMesh(core_axis_name="c", subcore_axis_name="s")

    def body(h2_hbm, ep_hbm, out_hbm, ebs, sb,
             rb0, rb1, rb2, acc_sh, isems, gsems, ssems):
        bufl = [rb0, rb1, rb2]
        c = lax.axis_index("c")
        s = lax.axis_index("s")

        zero16 = jnp.zeros((16,), jnp.float32)

        @pl.loop(0, CHUNK)
        def _(r):
            for k in range(HALF // 16):
                rb0[r, pl.ds(k * 16, 16)] = zero16

        base = s * ROWS_PER_SUB
        for t in range(4):
            pltpu.sync_copy(
                rb0, acc_sh.at[pl.ds(base + t * CHUNK, CHUNK)])

        @pl.when(s < NS - 1)
        def _():
            pltpu.sync_copy(
                rb0.at[pl.ds(0, ROWS_PER_SUB - 4 * CHUNK)],
                acc_sh.at[pl.ds(base + 4 * CHUNK, ROWS_PER_SUB - 4 * CHUNK)])

        @pl.when(s == NS - 1)
        def _():
            pltpu.sync_copy(
                rb0.at[pl.ds(0, ROWS_LAST - 4 * CHUNK)],
                acc_sh.at[pl.ds(base + 4 * CHUNK, ROWS_LAST - 4 * CHUNK)])

        plsc.subcore_barrier()

        def issue_idx(n, er):
            pltpu.async_copy(
                ep_hbm.at[c, s, n], ebs.at[pl.ds(er * 3, 3)], isems.at[er])

        def wait_idx(er):
            pltpu.make_async_copy(
                ep_hbm.at[c, s, 0], ebs.at[pl.ds(er * 3, 3)],
                isems.at[er]).wait()

        def issue_gather(er, br):
            pltpu.async_copy(
                h2_hbm.at[ebs.at[er * 3]], bufl[br], gsems.at[br])

        def wait_gather(br):
            pltpu.make_async_copy(
                h2_hbm.at[ebs.at[0]], bufl[br], gsems.at[br]).wait()

        def issue_scatter(br):
            pltpu.async_copy(
                bufl[br], acc_sh.at[sb.at[br]], ssems.at[br],
                add=True)

        def wait_scatter(br):
            pltpu.make_async_copy(
                bufl[br], acc_sh.at[sb.at[0]], ssems.at[br]).wait()

        def stage_dst(er, br):
            for kk in range(HALF // 16):
                sl = pl.ds(kk * 16, 16)
                sb[br, sl] = ebs[er * 3 + 1, sl]

        def scale(er, br):
            buf = bufl[br]

            @pl.loop(0, CHUNK, step=16)
            def _(e0):
                wv = plsc.bitcast(
                    ebs[er * 3 + 2, pl.ds(e0, 16)], jnp.float32)
                for i in range(16):
                    w = wv[i]
                    for kk in range(HALF // 16):
                        sl = pl.ds(kk * 16, 16)
                        buf[e0 + i, sl] = buf[e0 + i, sl] * w

        issue_idx(0, 0)
        issue_idx(1, 1)
        wait_idx(0)
        issue_gather(0, 0)

        @pl.loop(0, NCHUNK, step=STEP)
        def _(m):
            for k in range(STEP):
                bs = k % RING
                es = k % IRING

                if k >= 2:
                    wait_scatter((k - 2) % RING)
                else:
                    @pl.when(m > 0)
                    def _():
                        wait_scatter((k - 2) % RING)

                @pl.when(m + k + 1 < NCHUNK)
                def _():
                    wait_idx((k + 1) % IRING)
                    issue_gather((k + 1) % IRING, (k + 1) % RING)

                wait_gather(bs)
                stage_dst(es, bs)
                scale(es, bs)
                issue_scatter(bs)

                @pl.when(m + k + 2 < NCHUNK)
                def _():
                    issue_idx(m + k + 2, es)

        wait_scatter((NCHUNK - 2) % RING)
        wait_scatter((NCHUNK - 1) % RING)

        plsc.subcore_barrier()

        @pl.when(s < NS - 1)
        def _():
            pltpu.sync_copy(
                acc_sh.at[pl.ds(base, ROWS_PER_SUB)],
                out_hbm.at[c, pl.ds(base, ROWS_PER_SUB)])

        @pl.when(s == NS - 1)
        def _():
            pltpu.sync_copy(
                acc_sh.at[pl.ds(base, ROWS_LAST)],
                out_hbm.at[c, pl.ds(base, ROWS_LAST)])

    cp = pltpu.CompilerParams()
    if "needs_layout_passes" in pltpu.CompilerParams.__dataclass_fields__:
        cp = dataclasses.replace(cp, needs_layout_passes=False)
    run = pl.kernel(
        body,
        out_type=jax.ShapeDtypeStruct((NC, N_NODES, HALF), jnp.float32),
        mesh=mesh,
        compiler_params=cp,
        scratch_types=(
            [pltpu.VMEM((3 * IRING, CHUNK), jnp.int32)]
            + [pltpu.VMEM((RING, CHUNK), jnp.int32)]
            + [pltpu.VMEM((CHUNK, HALF), jnp.float32)] * RING
            + [
                pltpu.VMEM_SHARED((N_NODES, HALF), jnp.float32),
                pltpu.SemaphoreType.DMA((IRING,)),
                pltpu.SemaphoreType.DMA((RING,)),
                pltpu.SemaphoreType.DMA((RING,)),
            ]
        ),
    )
    return run(h2, epack)


def _mm_root(x, w_root):
    def body(x_ref, w_ref, o_ref):
        o_ref[...] = jnp.dot(x_ref[...], w_ref[...],
                             preferred_element_type=jnp.float32)

    return pl.pallas_call(
        body,
        grid=(GRID,),
        in_specs=[pl.BlockSpec((BLK, D), lambda i: (i, 0)),
                  pl.BlockSpec((D, D), lambda i: (0, 0))],
        out_specs=pl.BlockSpec((BLK, D), lambda i: (i, 0)),
        out_shape=jax.ShapeDtypeStruct((N_NODES, D), jnp.float32),
    )(x, w_root)


def _combine1(agg, r1, w_rel, b2, w_root):
    def body(a_ref, r1_ref, wr_ref, b_ref, wroot_ref, h_ref, r2_ref):
        z = (jnp.dot(a_ref[0], wr_ref[0:HALF, :],
                     preferred_element_type=jnp.float32)
             + jnp.dot(a_ref[1], wr_ref[HALF:D, :],
                       preferred_element_type=jnp.float32)
             + b_ref[...] + r1_ref[...])
        h = jnp.where(z >= 0, z, NEG_SLOPE * z)
        h_ref[...] = h
        r2_ref[...] = jnp.dot(h, wroot_ref[...],
                              preferred_element_type=jnp.float32)

    return pl.pallas_call(
        body,
        grid=(GRID,),
        in_specs=[pl.BlockSpec((NC, BLK, HALF), lambda i: (0, i, 0)),
                  pl.BlockSpec((BLK, D), lambda i: (i, 0)),
                  pl.BlockSpec((D, D), lambda i: (0, 0)),
                  pl.BlockSpec((1, D), lambda i: (0, 0)),
                  pl.BlockSpec((D, D), lambda i: (0, 0))],
        out_specs=[pl.BlockSpec((BLK, D), lambda i: (i, 0)),
                   pl.BlockSpec((BLK, D), lambda i: (i, 0))],
        out_shape=[jax.ShapeDtypeStruct((N_NODES, D), jnp.float32),
                   jax.ShapeDtypeStruct((N_NODES, D), jnp.float32)],
    )(agg, r1, w_rel, b2, w_root)


def _combine2(agg, r2, x, h1, w_rel, b2):
    def body(a_ref, r2_ref, x_ref, h1_ref, wr_ref, b_ref, o_ref):
        z = (jnp.dot(a_ref[0], wr_ref[0:HALF, :],
                     preferred_element_type=jnp.float32)
             + jnp.dot(a_ref[1], wr_ref[HALF:D, :],
                       preferred_element_type=jnp.float32)
             + b_ref[...] + r2_ref[...])
        h2 = jnp.where(z >= 0, z, NEG_SLOPE * z)
        o_ref[...] = (x_ref[...] + h1_ref[...] + h2) * (1.0 / 3.0)

    return pl.pallas_call(
        body,
        grid=(GRID,),
        in_specs=[pl.BlockSpec((NC, BLK, HALF), lambda i: (0, i, 0)),
                  pl.BlockSpec((BLK, D), lambda i: (i, 0)),
                  pl.BlockSpec((BLK, D), lambda i: (i, 0)),
                  pl.BlockSpec((BLK, D), lambda i: (i, 0)),
                  pl.BlockSpec((D, D), lambda i: (0, 0)),
                  pl.BlockSpec((1, D), lambda i: (0, 0))],
        out_specs=pl.BlockSpec((BLK, D), lambda i: (i, 0)),
        out_shape=jax.ShapeDtypeStruct((N_NODES, D), jnp.float32),
    )(agg, r2, x, h1, w_rel, b2)


def kernel(x, edge_index, edge_weight, W_rel, b_rel, W_root):
    src = edge_index[0]
    dst = edge_index[1]
    pad = EDGES_PAD - N_EDGES
    shp = (NS, NCHUNK, CHUNK)
    g0 = jnp.pad(src * 2, (0, pad)).reshape(shp)
    dsti = jnp.pad(dst, (0, pad)).reshape(shp)
    ewb = jnp.pad(edge_weight, (0, pad)).reshape(shp).view(jnp.int32)
    epack = jnp.stack([
        jnp.stack([g0, dsti, ewb], axis=2),
        jnp.stack([g0 + 1, dsti, ewb], axis=2),
    ])
    b2 = b_rel.reshape(1, D)

    r1 = _mm_root(x, W_root)
    agg1 = _seg_sum_sc(x.reshape(2 * N_NODES, HALF), epack)
    h1, r2 = _combine1(agg1, r1, W_rel, b2, W_root)
    agg2 = _seg_sum_sc(h1.reshape(2 * N_NODES, HALF), epack)
    out = _combine2(agg2, r2, x, h1, W_rel, b2)
    return out


# speedup vs baseline: 1.4279x; 1.4279x over previous
import dataclasses
import functools

import jax
import jax.numpy as jnp
from jax import lax
from jax.experimental import pallas as pl
from jax.experimental.pallas import tpu as pltpu
from jax.experimental.pallas import tpu_sc as plsc

N_NODES = 10000
N_EDGES = 160000
D = 256
HALF = 128
NEG_SLOPE = 0.2

NC = 2
NS = 16
CHUNK = 128
NCHUNK = 80
EDGES_PAD = NS * NCHUNK * CHUNK
N_PAD = 10112
ROWS_PER_SUB = N_PAD // NS

BLK = 1000
GRID = N_NODES // BLK


def _seg_sum_sc(h2, epack):
    mesh = plsc.VectorSubcoreMesh(core_axis_name="c", subcore_axis_name="s")

    def body(h2_hbm, ep_hbm, out_hbm, eb0, eb1, buf0, buf1, acc_sh,
             isem0, isem1, gsem0, gsem1):
        c = lax.axis_index("c")
        s = lax.axis_index("s")

        zero16 = jnp.zeros((16,), jnp.float32)

        @pl.loop(0, CHUNK)
        def _(r):
            for k in range(HALF // 16):
                buf0[r, pl.ds(k * 16, 16)] = zero16

        for t in range(ROWS_PER_SUB // CHUNK):
            pltpu.sync_copy(
                buf0, acc_sh.at[pl.ds(s * ROWS_PER_SUB + t * CHUNK, CHUNK)])
        rem = ROWS_PER_SUB % CHUNK
        if rem:
            pltpu.sync_copy(
                buf0.at[pl.ds(0, rem)],
                acc_sh.at[pl.ds(s * ROWS_PER_SUB + ROWS_PER_SUB - rem, rem)])
        plsc.subcore_barrier()

        def issue_idx(j, eb, isem):
            pltpu.async_copy(ep_hbm.at[c, s, j], eb, isem)

        def wait_idx(eb, isem):
            pltpu.make_async_copy(ep_hbm.at[c, s, 0], eb, isem).wait()

        def issue_gather(eb, buf, gsem):
            pltpu.async_copy(h2_hbm.at[eb.at[0]], buf, gsem)

        def wait_gather(buf, gsem):
            pltpu.make_async_copy(h2_hbm.at[eb0.at[0]], buf, gsem).wait()

        def process(eb, buf):
            @pl.loop(0, CHUNK, step=16)
            def _(e0):
                wv = plsc.bitcast(eb[2, pl.ds(e0, 16)], jnp.float32)
                for i in range(16):
                    w = wv[i]
                    for k in range(HALF // 16):
                        sl = pl.ds(k * 16, 16)
                        buf[e0 + i, sl] = buf[e0 + i, sl] * w

            pltpu.sync_copy(buf, acc_sh.at[eb.at[1]], add=True)

        issue_idx(0, eb0, isem0)
        wait_idx(eb0, isem0)
        issue_gather(eb0, buf0, gsem0)
        issue_idx(1, eb1, isem1)

        @pl.loop(0, NCHUNK, step=2)
        def _(j):
            wait_idx(eb1, isem1)
            issue_gather(eb1, buf1, gsem1)
            wait_gather(buf0, gsem0)
            process(eb0, buf0)

            @pl.when(j + 2 < NCHUNK)
            def _():
                issue_idx(j + 2, eb0, isem0)

            @pl.when(j + 2 < NCHUNK)
            def _():
                wait_idx(eb0, isem0)
                issue_gather(eb0, buf0, gsem0)

            wait_gather(buf1, gsem1)
            process(eb1, buf1)

            @pl.when(j + 3 < NCHUNK)
            def _():
                issue_idx(j + 3, eb1, isem1)

        plsc.subcore_barrier()
        pltpu.sync_copy(
            acc_sh.at[pl.ds(s * ROWS_PER_SUB, ROWS_PER_SUB)],
            out_hbm.at[c, pl.ds(s * ROWS_PER_SUB, ROWS_PER_SUB)])

    cp = pltpu.CompilerParams()
    if "needs_layout_passes" in pltpu.CompilerParams.__dataclass_fields__:
        cp = dataclasses.replace(cp, needs_layout_passes=False)
    run = pl.kernel(
        body,
        out_type=jax.ShapeDtypeStruct((NC, N_PAD, HALF), jnp.float32),
        mesh=mesh,
        compiler_params=cp,
        scratch_types=[
            pltpu.VMEM((3, CHUNK), jnp.int32),
            pltpu.VMEM((3, CHUNK), jnp.int32),
            pltpu.VMEM((CHUNK, HALF), jnp.float32),
            pltpu.VMEM((CHUNK, HALF), jnp.float32),
            pltpu.VMEM_SHARED((N_PAD, HALF), jnp.float32),
            pltpu.SemaphoreType.DMA,
            pltpu.SemaphoreType.DMA,
            pltpu.SemaphoreType.DMA,
            pltpu.SemaphoreType.DMA,
        ],
    )
    return run(h2, epack)


def _mm_root(x, w_root):
    def body(x_ref, w_ref, o_ref):
        o_ref[...] = jnp.dot(x_ref[...], w_ref[...],
                             preferred_element_type=jnp.float32)

    return pl.pallas_call(
        body,
        grid=(GRID,),
        in_specs=[pl.BlockSpec((BLK, D), lambda i: (i, 0)),
                  pl.BlockSpec((D, D), lambda i: (0, 0))],
        out_specs=pl.BlockSpec((BLK, D), lambda i: (i, 0)),
        out_shape=jax.ShapeDtypeStruct((N_NODES, D), jnp.float32),
    )(x, w_root)


def _combine1(agg, r1, w_rel, b2, w_root):
    def body(a_ref, r1_ref, wr_ref, b_ref, wroot_ref, h_ref, r2_ref):
        z = (jnp.dot(a_ref[0], wr_ref[0:HALF, :],
                     preferred_element_type=jnp.float32)
             + jnp.dot(a_ref[1], wr_ref[HALF:D, :],
                       preferred_element_type=jnp.float32)
             + b_ref[...] + r1_ref[...])
        h = jnp.where(z >= 0, z, NEG_SLOPE * z)
        h_ref[...] = h
        r2_ref[...] = jnp.dot(h, wroot_ref[...],
                              preferred_element_type=jnp.float32)

    return pl.pallas_call(
        body,
        grid=(GRID,),
        in_specs=[pl.BlockSpec((NC, BLK, HALF), lambda i: (0, i, 0)),
                  pl.BlockSpec((BLK, D), lambda i: (i, 0)),
                  pl.BlockSpec((D, D), lambda i: (0, 0)),
                  pl.BlockSpec((1, D), lambda i: (0, 0)),
                  pl.BlockSpec((D, D), lambda i: (0, 0))],
        out_specs=[pl.BlockSpec((BLK, D), lambda i: (i, 0)),
                   pl.BlockSpec((BLK, D), lambda i: (i, 0))],
        out_shape=[jax.ShapeDtypeStruct((N_NODES, D), jnp.float32),
                   jax.ShapeDtypeStruct((N_NODES, D), jnp.float32)],
    )(agg, r1, w_rel, b2, w_root)


def _combine2(agg, r2, x, h1, w_rel, b2):
    def body(a_ref, r2_ref, x_ref, h1_ref, wr_ref, b_ref, o_ref):
        z = (jnp.dot(a_ref[0], wr_ref[0:HALF, :],
                     preferred_element_type=jnp.float32)
             + jnp.dot(a_ref[1], wr_ref[HALF:D, :],
                       preferred_element_type=jnp.float32)
             + b_ref[...] + r2_ref[...])
        h2 = jnp.where(z >= 0, z, NEG_SLOPE * z)
        o_ref[...] = (x_ref[...] + h1_ref[...] + h2) * (1.0 / 3.0)

    return pl.pallas_call(
        body,
        grid=(GRID,),
        in_specs=[pl.BlockSpec((NC, BLK, HALF), lambda i: (0, i, 0)),
                  pl.BlockSpec((BLK, D), lambda i: (i, 0)),
                  pl.BlockSpec((BLK, D), lambda i: (i, 0)),
                  pl.BlockSpec((BLK, D), lambda i: (i, 0)),
                  pl.BlockSpec((D, D), lambda i: (0, 0)),
                  pl.BlockSpec((1, D), lambda i: (0, 0))],
        out_specs=pl.BlockSpec((BLK, D), lambda i: (i, 0)),
        out_shape=jax.ShapeDtypeStruct((N_NODES, D), jnp.float32),
    )(agg, r2, x, h1, w_rel, b2)


def kernel(x, edge_index, edge_weight, W_rel, b_rel, W_root):
    order = jnp.argsort(edge_index[0])
    src = edge_index[0][order]
    dst = edge_index[1][order]
    edge_weight = edge_weight[order]
    pad = EDGES_PAD - N_EDGES
    shp = (NS, NCHUNK, CHUNK)
    g0 = jnp.pad(src * 2, (0, pad)).reshape(shp)
    dsti = jnp.pad(dst, (0, pad)).reshape(shp)
    ewb = jnp.pad(edge_weight, (0, pad)).reshape(shp).view(jnp.int32)
    epack = jnp.stack([
        jnp.stack([g0, dsti, ewb], axis=2),
        jnp.stack([g0 + 1, dsti, ewb], axis=2),
    ])
    b2 = b_rel.reshape(1, D)

    r1 = _mm_root(x, W_root)
    agg1 = _seg_sum_sc(x.reshape(2 * N_NODES, HALF), epack)
    h1, r2 = _combine1(agg1, r1, W_rel, b2, W_root)
    agg2 = _seg_sum_sc(h1.reshape(2 * N_NODES, HALF), epack)
    out = _combine2(agg2, r2, x, h1, W_rel, b2)
    return out


# speedup vs baseline: 1.8705x; 1.3099x over previous
import dataclasses
import functools

import jax
import jax.numpy as jnp
from jax import lax
from jax.experimental import pallas as pl
from jax.experimental.pallas import tpu as pltpu
from jax.experimental.pallas import tpu_sc as plsc

N_NODES = 10000
N_EDGES = 160000
D = 256
HALF = 128
NEG_SLOPE = 0.2

NC = 2
NS = 16
CHUNK = 128
NCHUNK = 80
EDGES_PAD = NS * NCHUNK * CHUNK
N_PAD = 10112
ROWS_PER_SUB = N_PAD // NS

BLK = 1000
GRID = N_NODES // BLK


def _seg_sum_sc(h2, epack):
    mesh = plsc.VectorSubcoreMesh(core_axis_name="c", subcore_axis_name="s")

    def body(h2_hbm, ep_hbm, out_hbm, eb0, eb1, buf0, buf1, acc_sh,
             isem0, isem1, gsem0, gsem1):
        c = lax.axis_index("c")
        s = lax.axis_index("s")

        zero16 = jnp.zeros((16,), jnp.float32)

        @pl.loop(0, CHUNK)
        def _(r):
            for k in range(HALF // 16):
                buf0[r, pl.ds(k * 16, 16)] = zero16

        for t in range(ROWS_PER_SUB // CHUNK):
            pltpu.sync_copy(
                buf0, acc_sh.at[pl.ds(s * ROWS_PER_SUB + t * CHUNK, CHUNK)])
        rem = ROWS_PER_SUB % CHUNK
        if rem:
            pltpu.sync_copy(
                buf0.at[pl.ds(0, rem)],
                acc_sh.at[pl.ds(s * ROWS_PER_SUB + ROWS_PER_SUB - rem, rem)])
        plsc.subcore_barrier()

        def issue_idx(j, eb, isem):
            pltpu.async_copy(ep_hbm.at[c, s, j], eb, isem)

        def wait_idx(eb, isem):
            pltpu.make_async_copy(ep_hbm.at[c, s, 0], eb, isem).wait()

        def issue_gather(eb, buf, gsem):
            pltpu.async_copy(h2_hbm.at[eb.at[0]], buf, gsem)

        def wait_gather(buf, gsem):
            pltpu.make_async_copy(h2_hbm.at[eb0.at[0]], buf, gsem).wait()

        def process(eb, buf):
            @pl.loop(0, CHUNK, step=16)
            def _(e0):
                wv = plsc.bitcast(eb[2, pl.ds(e0, 16)], jnp.float32)
                for i in range(16):
                    w = wv[i]
                    for k in range(HALF // 16):
                        sl = pl.ds(k * 16, 16)
                        buf[e0 + i, sl] = buf[e0 + i, sl] * w

            pltpu.sync_copy(buf, acc_sh.at[eb.at[1]], add=True)

        issue_idx(0, eb0, isem0)
        wait_idx(eb0, isem0)
        issue_gather(eb0, buf0, gsem0)
        issue_idx(1, eb1, isem1)

        @pl.loop(0, NCHUNK, step=2)
        def _(j):
            wait_idx(eb1, isem1)
            issue_gather(eb1, buf1, gsem1)
            wait_gather(buf0, gsem0)
            process(eb0, buf0)

            @pl.when(j + 2 < NCHUNK)
            def _():
                issue_idx(j + 2, eb0, isem0)

            @pl.when(j + 2 < NCHUNK)
            def _():
                wait_idx(eb0, isem0)
                issue_gather(eb0, buf0, gsem0)

            wait_gather(buf1, gsem1)
            process(eb1, buf1)

            @pl.when(j + 3 < NCHUNK)
            def _():
                issue_idx(j + 3, eb1, isem1)

        plsc.subcore_barrier()
        pltpu.sync_copy(
            acc_sh.at[pl.ds(s * ROWS_PER_SUB, ROWS_PER_SUB)],
            out_hbm.at[c, pl.ds(s * ROWS_PER_SUB, ROWS_PER_SUB)])

    cp = pltpu.CompilerParams()
    if "needs_layout_passes" in pltpu.CompilerParams.__dataclass_fields__:
        cp = dataclasses.replace(cp, needs_layout_passes=False)
    run = pl.kernel(
        body,
        out_type=jax.ShapeDtypeStruct((NC, N_PAD, HALF), jnp.float32),
        mesh=mesh,
        compiler_params=cp,
        scratch_types=[
            pltpu.VMEM((3, CHUNK), jnp.int32),
            pltpu.VMEM((3, CHUNK), jnp.int32),
            pltpu.VMEM((CHUNK, HALF), jnp.float32),
            pltpu.VMEM((CHUNK, HALF), jnp.float32),
            pltpu.VMEM_SHARED((N_PAD, HALF), jnp.float32),
            pltpu.SemaphoreType.DMA,
            pltpu.SemaphoreType.DMA,
            pltpu.SemaphoreType.DMA,
            pltpu.SemaphoreType.DMA,
        ],
    )
    return run(h2, epack)


def _mm_root(x, w_root):
    def body(x_ref, w_ref, o_ref):
        o_ref[...] = jnp.dot(x_ref[...], w_ref[...],
                             preferred_element_type=jnp.float32)

    return pl.pallas_call(
        body,
        grid=(GRID,),
        in_specs=[pl.BlockSpec((BLK, D), lambda i: (i, 0)),
                  pl.BlockSpec((D, D), lambda i: (0, 0))],
        out_specs=pl.BlockSpec((BLK, D), lambda i: (i, 0)),
        out_shape=jax.ShapeDtypeStruct((N_NODES, D), jnp.float32),
    )(x, w_root)


def _combine1(agg, r1, w_rel, b2, w_root):
    def body(a_ref, r1_ref, wr_ref, b_ref, wroot_ref, h_ref, r2_ref):
        z = (jnp.dot(a_ref[0], wr_ref[0:HALF, :],
                     preferred_element_type=jnp.float32)
             + jnp.dot(a_ref[1], wr_ref[HALF:D, :],
                       preferred_element_type=jnp.float32)
             + b_ref[...] + r1_ref[...])
        h = jnp.where(z >= 0, z, NEG_SLOPE * z)
        h_ref[...] = h
        r2_ref[...] = jnp.dot(h, wroot_ref[...],
                              preferred_element_type=jnp.float32)

    return pl.pallas_call(
        body,
        grid=(GRID,),
        in_specs=[pl.BlockSpec((NC, BLK, HALF), lambda i: (0, i, 0)),
                  pl.BlockSpec((BLK, D), lambda i: (i, 0)),
                  pl.BlockSpec((D, D), lambda i: (0, 0)),
                  pl.BlockSpec((1, D), lambda i: (0, 0)),
                  pl.BlockSpec((D, D), lambda i: (0, 0))],
        out_specs=[pl.BlockSpec((BLK, D), lambda i: (i, 0)),
                   pl.BlockSpec((BLK, D), lambda i: (i, 0))],
        out_shape=[jax.ShapeDtypeStruct((N_NODES, D), jnp.float32),
                   jax.ShapeDtypeStruct((N_NODES, D), jnp.float32)],
    )(agg, r1, w_rel, b2, w_root)


def _combine2(agg, r2, x, h1, w_rel, b2):
    def body(a_ref, r2_ref, x_ref, h1_ref, wr_ref, b_ref, o_ref):
        z = (jnp.dot(a_ref[0], wr_ref[0:HALF, :],
                     preferred_element_type=jnp.float32)
             + jnp.dot(a_ref[1], wr_ref[HALF:D, :],
                       preferred_element_type=jnp.float32)
             + b_ref[...] + r2_ref[...])
        h2 = jnp.where(z >= 0, z, NEG_SLOPE * z)
        o_ref[...] = (x_ref[...] + h1_ref[...] + h2) * (1.0 / 3.0)

    return pl.pallas_call(
        body,
        grid=(GRID,),
        in_specs=[pl.BlockSpec((NC, BLK, HALF), lambda i: (0, i, 0)),
                  pl.BlockSpec((BLK, D), lambda i: (i, 0)),
                  pl.BlockSpec((BLK, D), lambda i: (i, 0)),
                  pl.BlockSpec((BLK, D), lambda i: (i, 0)),
                  pl.BlockSpec((D, D), lambda i: (0, 0)),
                  pl.BlockSpec((1, D), lambda i: (0, 0))],
        out_specs=pl.BlockSpec((BLK, D), lambda i: (i, 0)),
        out_shape=jax.ShapeDtypeStruct((N_NODES, D), jnp.float32),
    )(agg, r2, x, h1, w_rel, b2)


def kernel(x, edge_index, edge_weight, W_rel, b_rel, W_root):
    src = edge_index[0]
    dst = edge_index[1]
    pad = EDGES_PAD - N_EDGES
    shp = (NS, NCHUNK, CHUNK)
    g0 = jnp.pad(src * 2, (0, pad)).reshape(shp)
    dsti = jnp.pad(dst, (0, pad)).reshape(shp)
    ewb = jnp.pad(edge_weight, (0, pad)).reshape(shp).view(jnp.int32)
    epack = jnp.stack([
        jnp.stack([g0, dsti, ewb], axis=2),
        jnp.stack([g0 + 1, dsti, ewb], axis=2),
    ])
    b2 = b_rel.reshape(1, D)

    r1 = _mm_root(x, W_root)
    agg1 = _seg_sum_sc(x.reshape(2 * N_NODES, HALF), epack)
    h1, r2 = _combine1(agg1, r1, W_rel, b2, W_root)
    agg2 = _seg_sum_sc(h1.reshape(2 * N_NODES, HALF), epack)
    out = _combine2(agg2, r2, x, h1, W_rel, b2)
    return out
